# Initial kernel scaffold; baseline (speedup 1.0000x reference)
import jax
import jax.numpy as jnp
from jax.experimental import pallas as pl


def kernel(x):
    raise NotImplementedError("write your pallas kernel here")


# speedup vs baseline: 1.3705x; 1.3705x over previous
import jax
import jax.numpy as jnp
from jax.experimental import pallas as pl

_NUM_ITEMS = 100000
_COL_BLOCK = 1024


def _onehot_body(it_ref, o_ref):
    j = pl.program_id(0)
    cols = jax.lax.broadcasted_iota(jnp.int32, o_ref.shape, 1) + j * o_ref.shape[1]
    o_ref[...] = (cols == it_ref[...]).astype(jnp.float32)


def kernel(x):
    B = x.shape[0]
    it = jax.random.randint(jax.random.key(42), (B,), 1, _NUM_ITEMS)
    it2 = it.astype(jnp.int32).reshape(B, 1)
    grid = (pl.cdiv(_NUM_ITEMS, _COL_BLOCK),)
    out = pl.pallas_call(
        _onehot_body,
        grid=grid,
        in_specs=[pl.BlockSpec((B, 1), lambda j: (0, 0))],
        out_specs=pl.BlockSpec((B, _COL_BLOCK), lambda j: (0, j)),
        out_shape=jax.ShapeDtypeStruct((B, _NUM_ITEMS), jnp.float32),
    )(it2)
    return out
